# Initial kernel scaffold; baseline (speedup 1.0000x reference)
import functools

import jax
import jax.numpy as jnp
from jax import lax
from jax.experimental import pallas as pl
from jax.experimental.pallas import tpu as pltpu
from jax.experimental.pallas import tpu_sc as plsc

N = 10000
NP = 10240
E = 320000
D = 128
C = 16

NC = 2
NS = 16
EW = E // (NC * NS)
EB = 80
NB = EW // EB
SLAB = NP // NS
DEG_EW = E // NS
DEG_NB = DEG_EW // EB

_MESH = plsc.VectorSubcoreMesh(core_axis_name="c", subcore_axis_name="s")


@functools.partial(
    pl.kernel,
    out_type=jax.ShapeDtypeStruct((NC, NP), jnp.float32),
    mesh=_MESH,
    scratch_types=[
        pltpu.VMEM_SHARED((NP,), jnp.float32),
        pltpu.VMEM((SLAB,), jnp.float32),
        pltpu.VMEM((EB,), jnp.int32),
        pltpu.VMEM((EB,), jnp.float32),
    ],
)
def _sc_degrees(idx_hbm, out_hbm, acc_sh, slab_v, idx_v, ones_v):
    c = lax.axis_index("c")
    s = lax.axis_index("s")

    def _fill_slab(i, _):
        slab_v[pl.ds(i * 16, 16)] = jnp.zeros((16,), jnp.float32)
        return 0
    lax.fori_loop(0, SLAB // 16, _fill_slab, 0)

    def _fill_ones(i, _):
        ones_v[pl.ds(i * 16, 16)] = jnp.ones((16,), jnp.float32)
        return 0
    lax.fori_loop(0, EB // 16, _fill_ones, 0)

    base = s * SLAB
    pltpu.sync_copy(slab_v, acc_sh.at[pl.ds(base, SLAB)])
    plsc.subcore_barrier()

    def _batch(i, _):
        off = s * DEG_EW + i * EB
        pltpu.sync_copy(idx_hbm.at[c, pl.ds(off, EB)], idx_v)
        pltpu.sync_copy(ones_v, acc_sh.at[idx_v], add=True)
        return 0
    lax.fori_loop(0, DEG_NB, _batch, 0)
    plsc.subcore_barrier()

    pltpu.sync_copy(acc_sh.at[pl.ds(base, SLAB)], slab_v)
    pltpu.sync_copy(slab_v, out_hbm.at[c, pl.ds(base, SLAB)])


@functools.partial(
    pl.kernel,
    out_type=jax.ShapeDtypeStruct((NC, NP, D), jnp.float32),
    mesh=_MESH,
    scratch_types=[
        pltpu.VMEM_SHARED((NP, D), jnp.float32),
        pltpu.VMEM((128, D), jnp.float32),
        pltpu.VMEM((EB,), jnp.int32),
        pltpu.VMEM((EB,), jnp.int32),
        pltpu.VMEM((EB, D), jnp.float32),
        pltpu.SemaphoreType.DMA,
    ],
)
def _sc_aggregate(h_hbm, src_hbm, dst_hbm, out_hbm,
                  acc_sh, stage_v, src_v, dst_v, rows_v, sem):
    c = lax.axis_index("c")
    s = lax.axis_index("s")
    wid = c * NS + s

    def _zrow(i, _):
        def _zlane(j, _):
            stage_v[i, pl.ds(j * 16, 16)] = jnp.zeros((16,), jnp.float32)
            return 0
        lax.fori_loop(0, D // 16, _zlane, 0)
        return 0
    lax.fori_loop(0, 128, _zrow, 0)

    base = s * SLAB
    def _zslab(k, _):
        pltpu.sync_copy(stage_v, acc_sh.at[pl.ds(base + k * 128, 128)])
        return 0
    lax.fori_loop(0, SLAB // 128, _zslab, 0)
    plsc.subcore_barrier()

    ebase = wid * EW
    def _batch(i, _):
        off = ebase + i * EB
        pltpu.sync_copy(src_hbm.at[pl.ds(off, EB)], src_v)
        pltpu.sync_copy(dst_hbm.at[pl.ds(off, EB)], dst_v)
        pltpu.async_copy(h_hbm.at[src_v], rows_v, sem).wait()
        pltpu.sync_copy(rows_v, acc_sh.at[dst_v], add=True)
        return 0
    lax.fori_loop(0, NB, _batch, 0)
    plsc.subcore_barrier()

    def _wb(k, _):
        pltpu.sync_copy(acc_sh.at[pl.ds(base + k * 128, 128)], stage_v)
        pltpu.sync_copy(stage_v, out_hbm.at[c, pl.ds(base + k * 128, 128)])
        return 0
    lax.fori_loop(0, SLAB // 128, _wb, 0)


def _tc_pre_body(x_ref, w_ref, dout_ref, o_ref):
    scale = lax.rsqrt(jnp.maximum(dout_ref[...], 1.0))
    o_ref[...] = jnp.dot(x_ref[...], w_ref[...],
                         preferred_element_type=jnp.float32) * scale


def _tc_mid_body(p_ref, din_ref, b_ref, w_ref, dout_ref, o_ref):
    agg = p_ref[0] + p_ref[1]
    din = lax.rsqrt(jnp.maximum(din_ref[...], 1.0))
    x = jnp.maximum(agg * din + b_ref[...], 0.0)
    dout = lax.rsqrt(jnp.maximum(dout_ref[...], 1.0))
    o_ref[...] = jnp.dot(x, w_ref[...],
                         preferred_element_type=jnp.float32) * dout


def _tc_final_body(p_ref, din_ref, b_ref, wl_ref, bl_ref, o_ref):
    agg = p_ref[0] + p_ref[1]
    din = lax.rsqrt(jnp.maximum(din_ref[...], 1.0))
    x = agg * din + b_ref[...]
    logits = jnp.dot(x, wl_ref[...],
                     preferred_element_type=jnp.float32) + bl_ref[...]
    m = jnp.max(logits, axis=-1, keepdims=True)
    ex = jnp.exp(logits - m)
    o_ref[...] = ex / jnp.sum(ex, axis=-1, keepdims=True)


_tc_pre = pl.pallas_call(
    _tc_pre_body, out_shape=jax.ShapeDtypeStruct((NP, D), jnp.float32))
_tc_mid = pl.pallas_call(
    _tc_mid_body, out_shape=jax.ShapeDtypeStruct((NP, D), jnp.float32))
_tc_final = pl.pallas_call(
    _tc_final_body, out_shape=jax.ShapeDtypeStruct((NP, C), jnp.float32))


def kernel(in_feat, edge_index, W1, b1, W2, b2, W3, b3, Wl, bl):
    src = edge_index[0]
    dst = edge_index[1]
    degs = _sc_degrees(edge_index)
    dout = degs[0].reshape(NP, 1)
    din = degs[1].reshape(NP, 1)
    xp = jnp.pad(in_feat, ((0, NP - N), (0, 0)))
    h = _tc_pre(xp, W1, dout)
    p = _sc_aggregate(h, src, dst)
    h = _tc_mid(p, din, b1, W2, dout)
    p = _sc_aggregate(h, src, dst)
    h = _tc_mid(p, din, b2, W3, dout)
    p = _sc_aggregate(h, src, dst)
    out = _tc_final(p, din, b3, Wl, bl)
    return out[:N]


# speedup vs baseline: 4.5088x; 4.5088x over previous
import functools

import jax
import jax.numpy as jnp
from jax import lax
from jax.experimental import pallas as pl
from jax.experimental.pallas import tpu as pltpu
from jax.experimental.pallas import tpu_sc as plsc

N = 10000
NP = 10240
E = 320000
D = 128
C = 16

NC = 2
NS = 16
EW = E // (NC * NS)
EB = 80
NB = EW // EB
SLAB = NP // NS
DEG_EW = E // NS
DEG_NB = DEG_EW // EB

_MESH = plsc.VectorSubcoreMesh(core_axis_name="c", subcore_axis_name="s")


@functools.partial(
    pl.kernel,
    out_type=jax.ShapeDtypeStruct((NC, NP), jnp.float32),
    mesh=_MESH,
    scratch_types=[
        pltpu.VMEM_SHARED((NP,), jnp.float32),
        pltpu.VMEM((SLAB,), jnp.float32),
        pltpu.VMEM((EB,), jnp.int32),
        pltpu.VMEM((EB,), jnp.float32),
    ],
)
def _sc_degrees(idx_hbm, out_hbm, acc_sh, slab_v, idx_v, ones_v):
    c = lax.axis_index("c")
    s = lax.axis_index("s")

    def _fill_slab(i, _):
        slab_v[pl.ds(i * 16, 16)] = jnp.zeros((16,), jnp.float32)
        return 0
    lax.fori_loop(0, SLAB // 16, _fill_slab, 0)

    def _fill_ones(i, _):
        ones_v[pl.ds(i * 16, 16)] = jnp.ones((16,), jnp.float32)
        return 0
    lax.fori_loop(0, EB // 16, _fill_ones, 0)

    base = s * SLAB
    pltpu.sync_copy(slab_v, acc_sh.at[pl.ds(base, SLAB)])
    plsc.subcore_barrier()

    def _batch(i, _):
        off = c * E + s * DEG_EW + i * EB
        pltpu.sync_copy(idx_hbm.at[pl.ds(off, EB)], idx_v)
        pltpu.sync_copy(ones_v, acc_sh.at[idx_v], add=True)
        return 0
    lax.fori_loop(0, DEG_NB, _batch, 0)
    plsc.subcore_barrier()

    pltpu.sync_copy(acc_sh.at[pl.ds(base, SLAB)], slab_v)
    pltpu.sync_copy(slab_v, out_hbm.at[c, pl.ds(base, SLAB)])


@functools.partial(
    pl.kernel,
    out_type=jax.ShapeDtypeStruct((NC, NP, D), jnp.float32),
    mesh=_MESH,
    scratch_types=[
        pltpu.VMEM_SHARED((NP, D), jnp.float32),
        pltpu.VMEM((128, D), jnp.float32),
        pltpu.VMEM((EB,), jnp.int32),
        pltpu.VMEM((EB,), jnp.int32),
        pltpu.VMEM((EB, D), jnp.float32),
        pltpu.SemaphoreType.DMA,
    ],
)
def _sc_aggregate(h_hbm, src_hbm, dst_hbm, out_hbm,
                  acc_sh, stage_v, src_v, dst_v, rows_v, sem):
    c = lax.axis_index("c")
    s = lax.axis_index("s")
    wid = c * NS + s

    def _zrow(i, _):
        def _zlane(j, _):
            stage_v[i, pl.ds(j * 16, 16)] = jnp.zeros((16,), jnp.float32)
            return 0
        lax.fori_loop(0, D // 16, _zlane, 0)
        return 0
    lax.fori_loop(0, 128, _zrow, 0)

    base = s * SLAB
    def _zslab(k, _):
        pltpu.sync_copy(stage_v, acc_sh.at[pl.ds(base + k * 128, 128)])
        return 0
    lax.fori_loop(0, SLAB // 128, _zslab, 0)
    plsc.subcore_barrier()

    ebase = wid * EW
    def _batch(i, _):
        off = ebase + i * EB
        pltpu.sync_copy(src_hbm.at[pl.ds(off, EB)], src_v)
        pltpu.sync_copy(dst_hbm.at[pl.ds(off, EB)], dst_v)
        pltpu.async_copy(h_hbm.at[src_v], rows_v, sem).wait()
        pltpu.sync_copy(rows_v, acc_sh.at[dst_v], add=True)
        return 0
    lax.fori_loop(0, NB, _batch, 0)
    plsc.subcore_barrier()

    def _wb(k, _):
        pltpu.sync_copy(acc_sh.at[pl.ds(base + k * 128, 128)], stage_v)
        pltpu.sync_copy(stage_v, out_hbm.at[c, pl.ds(base + k * 128, 128)])
        return 0
    lax.fori_loop(0, SLAB // 128, _wb, 0)


def _tc_pre_body(x_ref, w_ref, dout_ref, o_ref):
    scale = lax.rsqrt(jnp.maximum(dout_ref[...], 1.0))
    o_ref[...] = jnp.dot(x_ref[...], w_ref[...],
                         preferred_element_type=jnp.float32) * scale


def _tc_mid_body(p_ref, din_ref, b_ref, w_ref, dout_ref, o_ref):
    agg = p_ref[0] + p_ref[1]
    din = lax.rsqrt(jnp.maximum(din_ref[...], 1.0))
    x = jnp.maximum(agg * din + b_ref[...], 0.0)
    dout = lax.rsqrt(jnp.maximum(dout_ref[...], 1.0))
    o_ref[...] = jnp.dot(x, w_ref[...],
                         preferred_element_type=jnp.float32) * dout


def _tc_final_body(p_ref, din_ref, b_ref, wl_ref, bl_ref, o_ref):
    agg = p_ref[0] + p_ref[1]
    din = lax.rsqrt(jnp.maximum(din_ref[...], 1.0))
    x = agg * din + b_ref[...]
    logits = jnp.dot(x, wl_ref[...],
                     preferred_element_type=jnp.float32) + bl_ref[...]
    m = jnp.max(logits, axis=-1, keepdims=True)
    ex = jnp.exp(logits - m)
    o_ref[...] = ex / jnp.sum(ex, axis=-1, keepdims=True)


_tc_pre = pl.pallas_call(
    _tc_pre_body, out_shape=jax.ShapeDtypeStruct((NP, D), jnp.float32))
_tc_mid = pl.pallas_call(
    _tc_mid_body, out_shape=jax.ShapeDtypeStruct((NP, D), jnp.float32))
_tc_final = pl.pallas_call(
    _tc_final_body, out_shape=jax.ShapeDtypeStruct((NP, C), jnp.float32))


def kernel(in_feat, edge_index, W1, b1, W2, b2, W3, b3, Wl, bl):
    src = edge_index[0]
    dst = edge_index[1]
    degs = _sc_degrees(edge_index.reshape(2 * E))
    dout = degs[0].reshape(NP, 1)
    din = degs[1].reshape(NP, 1)
    xp = jnp.pad(in_feat, ((0, NP - N), (0, 0)))
    h = _tc_pre(xp, W1, dout)
    p = _sc_aggregate(h, src, dst)
    h = _tc_mid(p, din, b1, W2, dout)
    p = _sc_aggregate(h, src, dst)
    h = _tc_mid(p, din, b2, W3, dout)
    p = _sc_aggregate(h, src, dst)
    out = _tc_final(p, din, b3, Wl, bl)
    return out[:N]
